# Initial kernel scaffold; baseline (speedup 1.0000x reference)
import functools

import jax
import jax.numpy as jnp
from jax import lax
from jax.experimental import pallas as pl
from jax.experimental.pallas import tpu as pltpu
from jax.experimental.pallas import tpu_sc as plsc

E = 320000
D = 128
NW = 32
EPW = E // NW
CH = 80
NCH = EPW // CH
G = CH // 16


def _body(z_hbm, src_hbm, dst_hbm, out_hbm,
          sidx, didx, srows, drows, obuf, sem_s, sem_d):
    c = lax.axis_index("c")
    s = lax.axis_index("s")
    wid = s * 2 + c
    base = wid * EPW

    def chunk_body(ci, carry):
        off = base + ci * CH
        pltpu.sync_copy(src_hbm.at[pl.ds(off, CH)], sidx)
        pltpu.sync_copy(dst_hbm.at[pl.ds(off, CH)], didx)
        cp_s = pltpu.async_copy(z_hbm.at[sidx], srows, sem_s)
        cp_d = pltpu.async_copy(z_hbm.at[didx], drows, sem_d)
        cp_s.wait()
        cp_d.wait()

        def grp_body(g, carry2):
            for jj in range(16):
                j = g * 16 + jj
                prods = [srows[j, pl.ds(k * 16, 16)] * drows[j, pl.ds(k * 16, 16)]
                         for k in range(8)]
                while len(prods) > 1:
                    prods = [prods[i] + prods[i + 1]
                             for i in range(0, len(prods), 2)]
                obuf[j] = jnp.sum(prods[0])
            v = obuf[pl.ds(g * 16, 16)]
            obuf[pl.ds(g * 16, 16)] = 1.0 / (1.0 + jnp.exp(-v))
            return carry2

        lax.fori_loop(0, G, grp_body, 0)
        pltpu.sync_copy(obuf, out_hbm.at[pl.ds(off, CH)])
        return carry

    lax.fori_loop(0, NCH, chunk_body, 0)


_mesh = plsc.VectorSubcoreMesh(core_axis_name="c", subcore_axis_name="s")

_sc_call = functools.partial(
    pl.kernel,
    out_type=jax.ShapeDtypeStruct((E,), jnp.float32),
    mesh=_mesh,
    scratch_types=[
        pltpu.VMEM((CH,), jnp.int32),
        pltpu.VMEM((CH,), jnp.int32),
        pltpu.VMEM((CH, D), jnp.float32),
        pltpu.VMEM((CH, D), jnp.float32),
        pltpu.VMEM((CH,), jnp.float32),
        pltpu.SemaphoreType.DMA,
        pltpu.SemaphoreType.DMA,
    ],
)(_body)


def kernel(z, edge_index):
    ei = edge_index.astype(jnp.int32)
    return _sc_call(z, ei[0], ei[1])


# speedup vs baseline: 2.7288x; 2.7288x over previous
import functools

import jax
import jax.numpy as jnp
from jax import lax
from jax.experimental import pallas as pl
from jax.experimental.pallas import tpu as pltpu
from jax.experimental.pallas import tpu_sc as plsc

E = 320000
D = 128
NW = 32
EPW = E // NW
CH = 80
NCH = EPW // CH
G = CH // 16


def _body(z_hbm, src_hbm, dst_hbm, out_hbm,
          sidx, didx, srows, drows, obuf, sem_s, sem_d):
    c = lax.axis_index("c")
    s = lax.axis_index("s")
    wid = s * 2 + c
    base = wid * EPW

    def chunk_body(ci, carry):
        off = base + ci * CH
        pltpu.sync_copy(src_hbm.at[pl.ds(off, CH)], sidx)
        pltpu.sync_copy(dst_hbm.at[pl.ds(off, CH)], didx)
        cp_s = pltpu.async_copy(z_hbm.at[sidx], srows, sem_s)
        cp_d = pltpu.async_copy(z_hbm.at[didx], drows, sem_d)
        cp_s.wait()
        cp_d.wait()

        iota16 = lax.iota(jnp.int32, 16)

        def grp_body(g, carry2):
            acc = jnp.zeros((16,), jnp.float32)
            for jj in range(16):
                j = g * 16 + jj
                prods = [srows[j, pl.ds(k * 16, 16)] * drows[j, pl.ds(k * 16, 16)]
                         for k in range(8)]
                while len(prods) > 1:
                    prods = [prods[i] + prods[i + 1]
                             for i in range(0, len(prods), 2)]
                tot = jnp.sum(prods[0])
                acc = jnp.where(iota16 == jj, tot, acc)
            obuf[pl.ds(g * 16, 16)] = 1.0 / (1.0 + jnp.exp(-acc))
            return carry2

        lax.fori_loop(0, G, grp_body, 0)
        pltpu.sync_copy(obuf, out_hbm.at[pl.ds(off, CH)])
        return carry

    lax.fori_loop(0, NCH, chunk_body, 0)


_mesh = plsc.VectorSubcoreMesh(core_axis_name="c", subcore_axis_name="s")

_sc_call = functools.partial(
    pl.kernel,
    out_type=jax.ShapeDtypeStruct((E,), jnp.float32),
    mesh=_mesh,
    compiler_params=pltpu.CompilerParams(needs_layout_passes=False),
    scratch_types=[
        pltpu.VMEM((CH,), jnp.int32),
        pltpu.VMEM((CH,), jnp.int32),
        pltpu.VMEM((CH, D), jnp.float32),
        pltpu.VMEM((CH, D), jnp.float32),
        pltpu.VMEM((CH,), jnp.float32),
        pltpu.SemaphoreType.DMA,
        pltpu.SemaphoreType.DMA,
    ],
)(_body)


def kernel(z, edge_index):
    ei = edge_index.astype(jnp.int32)
    return _sc_call(z, ei[0], ei[1])
